# baseline (speedup 1.0000x reference)
import jax
import jax.numpy as jnp
from jax import lax
from jax.experimental import pallas as pl
from jax.experimental.pallas import tpu as pltpu
from jax.experimental.pallas import tpu_sc as plsc

S = 64
NI = 128
NC = 2
NS = 16
NW = NC * NS
C = 128


def _srt(v):
    return lax.sort(v)


def _rev(v):
    return lax.rev(v, (0,))


def _bm2(x0, x1):
    return _srt(jnp.minimum(x0, x1)), _srt(jnp.maximum(x0, x1))


def _bm4(x0, x1, x2, x3):
    l0, l1 = jnp.minimum(x0, x2), jnp.minimum(x1, x3)
    h0, h1 = jnp.maximum(x0, x2), jnp.maximum(x1, x3)
    return _bm2(l0, l1) + _bm2(h0, h1)


def _m2(a, b):
    rb = _rev(b)
    return _bm2(jnp.minimum(a, rb), jnp.maximum(a, rb))


def _m4(a, b):
    rb0, rb1 = _rev(b[1]), _rev(b[0])
    return (_bm2(jnp.minimum(a[0], rb0), jnp.minimum(a[1], rb1))
            + _bm2(jnp.maximum(a[0], rb0), jnp.maximum(a[1], rb1)))


def _m8(a, b):
    rb = [_rev(b[3 - i]) for i in range(4)]
    l = [jnp.minimum(a[i], rb[i]) for i in range(4)]
    h = [jnp.maximum(a[i], rb[i]) for i in range(4)]
    return _bm4(*l) + _bm4(*h)


def _sc_kernel(dens_hbm, z_hbm, rays_hbm, u_hbm, out_hbm,
               dens_v, z_v, rays_v, u_v, out_v, cdf_f, bins_f):
    wid = lax.axis_index("s") * NC + lax.axis_index("c")
    rpw = u_hbm.shape[0] // NW
    base0 = wid * rpw
    iota = lax.iota(jnp.int32, 16)

    def chunk_body(ch, carry):
        base = base0 + ch * C
        pltpu.sync_copy(dens_hbm.at[:, pl.ds(base, C)], dens_v)
        pltpu.sync_copy(z_hbm.at[:, pl.ds(base, C)], z_v)
        pltpu.sync_copy(rays_hbm.at[pl.ds(base * 3, C * 3 + 16)], rays_v)
        pltpu.sync_copy(u_hbm.at[pl.ds(base, C)], u_v)

        def groupA(g, gc):
            rb = g * 16
            idx3 = (rb + iota) * 3
            rx = plsc.load_gather(rays_v, [idx3])
            ry = plsc.load_gather(rays_v, [idx3 + 1])
            rz = plsc.load_gather(rays_v, [idx3 + 2])
            nsq = rx * rx + ry * ry + rz * rz
            ib = (lax.bitcast_convert_type(nsq, jnp.int32) >> 1) + 0x1FBD1DF5
            nrm = lax.bitcast_convert_type(ib, jnp.float32)
            nrm = 0.5 * (nrm + nsq / nrm)
            nrm = 0.5 * (nrm + nsq / nrm)
            nrm = 0.5 * (nrm + nsq / nrm)
            scale = 100.0 * nrm

            acc = jnp.zeros((16,), jnp.float32)
            cacc = jnp.zeros((16,), jnp.float32)
            prev_z = z_v[0, pl.ds(rb, 16)]
            for j in range(S - 1):
                znx = z_v[j + 1, pl.ds(rb, 16)]
                bins_f[pl.ds(j * C + rb, 16)] = 0.5 * (prev_z + znx)
                x = jnp.maximum(dens_v[j, pl.ds(rb, 16)], 0.0) \
                    * ((znx - prev_z) * scale)
                prev_z = znx
                if j >= 1:
                    w = jnp.exp(-acc) * (1.0 - jnp.exp(-x))
                    cacc = cacc + (w + 1e-5)
                acc = acc + x
                cdf_f[pl.ds(j * C + rb, 16)] = cacc
            rec = 1.0 / cacc
            for j in range(S - 1):
                cdf_f[pl.ds(j * C + rb, 16)] = cdf_f[pl.ds(j * C + rb, 16)] * rec
            return gc

        lax.fori_loop(0, C // 16, groupA, 0)

        def rayB(r, rc):
            zouts = []
            for q in range(8):
                uu = u_v[r, pl.ds(16 * q, 16)]
                pos = jnp.zeros((16,), jnp.int32)
                for st in (32, 16, 8, 4, 2):
                    cand = pos + st
                    cv = plsc.load_gather(cdf_f, [(cand << 7) + r])
                    pos = jnp.where(cv <= uu, cand, pos)
                cand = pos + 1
                cv = plsc.load_gather(cdf_f, [(jnp.minimum(cand, 62) << 7) + r])
                pos = jnp.where((cand <= 62) & (cv <= uu), cand, pos)
                hi = jnp.minimum(pos + 1, 62)
                clo = plsc.load_gather(cdf_f, [(pos << 7) + r])
                chi = plsc.load_gather(cdf_f, [(hi << 7) + r])
                blo = plsc.load_gather(bins_f, [(pos << 7) + r])
                bhi = plsc.load_gather(bins_f, [(hi << 7) + r])
                den = chi - clo
                den = jnp.where(den < 1e-5, 1.0, den)
                t = (uu - clo) / den
                zouts.append(blo + t * (bhi - blo))

            v = [_srt(zq) for zq in zouts]
            fin = _m8(_m4(_m2(v[0], v[1]), _m2(v[2], v[3])),
                      _m4(_m2(v[4], v[5]), _m2(v[6], v[7])))
            for i in range(8):
                out_v[r, pl.ds(16 * i, 16)] = fin[i]
            return rc

        lax.fori_loop(0, C, rayB, 0)
        pltpu.sync_copy(out_v, out_hbm.at[pl.ds(base, C)])
        return carry

    lax.fori_loop(0, rpw // C, chunk_body, 0)


@jax.jit
def _run(dens_t, z_t, rays_flat, u):
    n = u.shape[0]
    mesh = plsc.VectorSubcoreMesh(core_axis_name="c", subcore_axis_name="s")
    f = pl.kernel(
        _sc_kernel,
        out_type=jax.ShapeDtypeStruct((n, NI), jnp.float32),
        mesh=mesh,
        compiler_params=pltpu.CompilerParams(needs_layout_passes=False),
        scratch_types=[
            pltpu.VMEM((S, C), jnp.float32),
            pltpu.VMEM((S, C), jnp.float32),
            pltpu.VMEM((C * 3 + 16,), jnp.float32),
            pltpu.VMEM((C, NI), jnp.float32),
            pltpu.VMEM((C, NI), jnp.float32),
            pltpu.VMEM(((S - 1) * C,), jnp.float32),
            pltpu.VMEM(((S - 1) * C,), jnp.float32),
        ],
    )
    return f(dens_t, z_t, rays_flat, u)


def kernel(density, z_vals, rays_d, u, N_importance):
    del N_importance
    rays_flat = jnp.concatenate(
        [rays_d.reshape(-1), jnp.zeros((16,), rays_d.dtype)])
    return _run(density[..., 0].T, z_vals.T, rays_flat, u)


# speedup vs baseline: 2.0868x; 2.0868x over previous
import jax
import jax.numpy as jnp
import numpy as np
from jax import lax
from jax.experimental import pallas as pl
from jax.experimental.pallas import tpu as pltpu
from jax.experimental.pallas import tpu_sc as plsc

S = 64
NI = 128


RB = 512
NROW = NI // 8


def _xor_shuffle8(c, j):
    if j == 4:
        return jnp.concatenate([c[4:8], c[0:4]], axis=0)
    if j == 2:
        return jnp.concatenate([c[2:4], c[0:2], c[6:8], c[4:6]], axis=0)
    return jnp.concatenate(
        [c[1:2], c[0:1], c[3:4], c[2:3], c[5:6], c[4:5], c[7:8], c[6:7]], axis=0)


def _sort128_sublanes(zs):
    rb = zs[0].shape[1]
    iota8 = jax.lax.broadcasted_iota(jnp.int32, (8, rb), 0)
    pat = {j: (iota8 & j) == 0 for j in (1, 2, 4)}
    mk = {(k, j): ((iota8 & j) == 0) ^ ((iota8 & k) != 0)
          for (k, j) in ((2, 1), (4, 2), (4, 1))}

    k = 2
    while k <= NI:
        j = k // 2
        while j >= 1:
            if j >= 8:
                jr = j // 8
                new = []
                for r in range(NROW):
                    a, b = zs[r], zs[r ^ jr]
                    take_lo = (((r * 8) & j) == 0) ^ (((r * 8) & k) != 0)
                    new.append(jnp.minimum(a, b) if take_lo else jnp.maximum(a, b))
                zs = new
            else:
                mask = mk[(k, j)] if k <= 4 else pat[j]
                for r in range(NROW):
                    c = zs[r]
                    p = _xor_shuffle8(c, j)
                    lo = jnp.minimum(c, p)
                    hi = jnp.maximum(c, p)
                    flip = ((r * 8) & k) != 0 if k >= 8 else False
                    zs[r] = jnp.where(mask, hi, lo) if flip else jnp.where(mask, lo, hi)
            j //= 2
        k *= 2
    return zs


def _tc_kernel(density_ref, z_ref, rays_ref, u_ref, m64_ref, m63_ref, out_ref):
    dens = density_ref[...]
    z_vals = z_ref[...]
    rays = rays_ref[...]
    m64 = m64_ref[...]
    m63 = m63_ref[...]

    norm = jnp.sqrt(jnp.sum(rays * rays, axis=0, keepdims=True))

    diffs = (z_vals[1:] - z_vals[:-1]) * 100.0
    dists = jnp.concatenate(
        [diffs, jnp.full((1, diffs.shape[1]), 1e10, dtype=diffs.dtype)], axis=0)
    dists = dists * norm

    x = jnp.maximum(dens, 0.0) * dists
    cx = jax.lax.dot_general(
        m64, x, (((1,), (0,)), ((), ())),
        precision=jax.lax.Precision.HIGHEST,
        preferred_element_type=jnp.float32)
    weights = jnp.exp(-cx) - jnp.exp(-(cx + x))

    w = weights[1:S - 1] + 1e-5
    tot = jnp.sum(w, axis=0, keepdims=True)
    cdf = jax.lax.dot_general(
        m63, w, (((1,), (0,)), ((), ())),
        precision=jax.lax.Precision.HIGHEST,
        preferred_element_type=jnp.float32) / tot

    bins = 0.5 * (z_vals[1:] + z_vals[:-1])

    denom = cdf[1:] - cdf[:-1]
    denom = jnp.where(denom < 1e-5, 1.0, denom)
    slope = (bins[1:] - bins[:-1]) / denom
    slope = jnp.concatenate(
        [slope, jnp.zeros((1, slope.shape[1]), dtype=slope.dtype)], axis=0)
    intercept = bins - cdf * slope

    u_all = u_ref[...]
    us = [u_all[8 * r:8 * r + 8] for r in range(NROW)]

    ab = jnp.broadcast_to(intercept[0:1], (8, RB))
    sb = jnp.broadcast_to(slope[0:1], (8, RB))
    zs = [ab + us[r] * sb for r in range(NROW)]
    for j in range(1, S - 1):
        cb = jnp.broadcast_to(cdf[j:j + 1], (8, RB))
        ab = jnp.broadcast_to(intercept[j:j + 1], (8, RB))
        sb = jnp.broadcast_to(slope[j:j + 1], (8, RB))
        for r in range(NROW):
            zs[r] = jnp.where(us[r] >= cb, ab + us[r] * sb, zs[r])

    zs = _sort128_sublanes(zs)
    out_ref[...] = jnp.concatenate(zs, axis=0)


def _run_tc(density_t, z_t, rays_t, u_t):
    n = density_t.shape[1]
    m64 = jnp.asarray(np.tril(np.ones((S, S), np.float32), -1))
    m63 = jnp.asarray(np.tril(np.ones((S - 1, S - 2), np.float32), -1))
    return pl.pallas_call(
        _tc_kernel,
        grid=(n // RB,),
        in_specs=[
            pl.BlockSpec((S, RB), lambda i: (0, i)),
            pl.BlockSpec((S, RB), lambda i: (0, i)),
            pl.BlockSpec((8, RB), lambda i: (0, i)),
            pl.BlockSpec((NI, RB), lambda i: (0, i)),
            pl.BlockSpec((S, S), lambda i: (0, 0)),
            pl.BlockSpec((S - 1, S - 2), lambda i: (0, 0)),
        ],
        out_specs=pl.BlockSpec((NI, RB), lambda i: (0, i)),
        out_shape=jax.ShapeDtypeStruct((NI, n), jnp.float32),
    )(density_t, z_t, rays_t, u_t, m64, m63)


NC = 2
NS = 16
NW = NC * NS
C = 128


def _srt(v):
    return lax.sort(v)


def _rev(v):
    return lax.rev(v, (0,))


def _bm2(x0, x1):
    return _srt(jnp.minimum(x0, x1)), _srt(jnp.maximum(x0, x1))


def _bm4(x0, x1, x2, x3):
    l0, l1 = jnp.minimum(x0, x2), jnp.minimum(x1, x3)
    h0, h1 = jnp.maximum(x0, x2), jnp.maximum(x1, x3)
    return _bm2(l0, l1) + _bm2(h0, h1)


def _m2(a, b):
    rb = _rev(b)
    return _bm2(jnp.minimum(a, rb), jnp.maximum(a, rb))


def _m4(a, b):
    rb0, rb1 = _rev(b[1]), _rev(b[0])
    return (_bm2(jnp.minimum(a[0], rb0), jnp.minimum(a[1], rb1))
            + _bm2(jnp.maximum(a[0], rb0), jnp.maximum(a[1], rb1)))


def _m8(a, b):
    rb = [_rev(b[3 - i]) for i in range(4)]
    l = [jnp.minimum(a[i], rb[i]) for i in range(4)]
    h = [jnp.maximum(a[i], rb[i]) for i in range(4)]
    return _bm4(*l) + _bm4(*h)


def _sc_kernel(dens_hbm, z_hbm, rays_hbm, u_hbm, out_hbm,
               dens_v, z_v, rays_v, u_v, out_v, cdf_f, bins_f):
    wid = lax.axis_index("s") * NC + lax.axis_index("c")
    rpw = u_hbm.shape[0] // NW
    base0 = wid * rpw
    iota = lax.iota(jnp.int32, 16)

    def chunk_body(ch, carry):
        base = base0 + ch * C
        pltpu.sync_copy(dens_hbm.at[:, pl.ds(base, C)], dens_v)
        pltpu.sync_copy(z_hbm.at[:, pl.ds(base, C)], z_v)
        pltpu.sync_copy(rays_hbm.at[pl.ds(base * 3, C * 3 + 16)], rays_v)
        pltpu.sync_copy(u_hbm.at[pl.ds(base, C)], u_v)

        def groupA(g, gc):
            rb = g * 16
            idx3 = (rb + iota) * 3
            rx = plsc.load_gather(rays_v, [idx3])
            ry = plsc.load_gather(rays_v, [idx3 + 1])
            rz = plsc.load_gather(rays_v, [idx3 + 2])
            nsq = rx * rx + ry * ry + rz * rz
            ib = (lax.bitcast_convert_type(nsq, jnp.int32) >> 1) + 0x1FBD1DF5
            nrm = lax.bitcast_convert_type(ib, jnp.float32)
            nrm = 0.5 * (nrm + nsq / nrm)
            nrm = 0.5 * (nrm + nsq / nrm)
            nrm = 0.5 * (nrm + nsq / nrm)
            scale = 100.0 * nrm

            acc = jnp.zeros((16,), jnp.float32)
            cacc = jnp.zeros((16,), jnp.float32)
            prev_z = z_v[0, pl.ds(rb, 16)]
            for j in range(S - 1):
                znx = z_v[j + 1, pl.ds(rb, 16)]
                bins_f[pl.ds(j * C + rb, 16)] = 0.5 * (prev_z + znx)
                x = jnp.maximum(dens_v[j, pl.ds(rb, 16)], 0.0) \
                    * ((znx - prev_z) * scale)
                prev_z = znx
                if j >= 1:
                    w = jnp.exp(-acc) * (1.0 - jnp.exp(-x))
                    cacc = cacc + (w + 1e-5)
                acc = acc + x
                cdf_f[pl.ds(j * C + rb, 16)] = cacc
            rec = 1.0 / cacc
            for j in range(S - 1):
                cdf_f[pl.ds(j * C + rb, 16)] = cdf_f[pl.ds(j * C + rb, 16)] * rec
            return gc

        lax.fori_loop(0, C // 16, groupA, 0)

        def rayB(r, rc):
            zouts = []
            for q in range(8):
                uu = u_v[r, pl.ds(16 * q, 16)]
                pos = jnp.zeros((16,), jnp.int32)
                for st in (32, 16, 8, 4, 2):
                    cand = pos + st
                    cv = plsc.load_gather(cdf_f, [(cand << 7) + r])
                    pos = jnp.where(cv <= uu, cand, pos)
                cand = pos + 1
                cv = plsc.load_gather(cdf_f, [(jnp.minimum(cand, 62) << 7) + r])
                pos = jnp.where((cand <= 62) & (cv <= uu), cand, pos)
                hi = jnp.minimum(pos + 1, 62)
                clo = plsc.load_gather(cdf_f, [(pos << 7) + r])
                chi = plsc.load_gather(cdf_f, [(hi << 7) + r])
                blo = plsc.load_gather(bins_f, [(pos << 7) + r])
                bhi = plsc.load_gather(bins_f, [(hi << 7) + r])
                den = chi - clo
                den = jnp.where(den < 1e-5, 1.0, den)
                t = (uu - clo) / den
                zouts.append(blo + t * (bhi - blo))

            v = [_srt(zq) for zq in zouts]
            fin = _m8(_m4(_m2(v[0], v[1]), _m2(v[2], v[3])),
                      _m4(_m2(v[4], v[5]), _m2(v[6], v[7])))
            for i in range(8):
                out_v[r, pl.ds(16 * i, 16)] = fin[i]
            return rc

        lax.fori_loop(0, C, rayB, 0)
        pltpu.sync_copy(out_v, out_hbm.at[pl.ds(base, C)])
        return carry

    lax.fori_loop(0, rpw // C, chunk_body, 0)


def _run_sc(dens_t, z_t, rays_flat, u):
    n = u.shape[0]
    mesh = plsc.VectorSubcoreMesh(core_axis_name="c", subcore_axis_name="s")
    f = pl.kernel(
        _sc_kernel,
        out_type=jax.ShapeDtypeStruct((n, NI), jnp.float32),
        mesh=mesh,
        compiler_params=pltpu.CompilerParams(needs_layout_passes=False),
        scratch_types=[
            pltpu.VMEM((S, C), jnp.float32),
            pltpu.VMEM((S, C), jnp.float32),
            pltpu.VMEM((C * 3 + 16,), jnp.float32),
            pltpu.VMEM((C, NI), jnp.float32),
            pltpu.VMEM((C, NI), jnp.float32),
            pltpu.VMEM(((S - 1) * C,), jnp.float32),
            pltpu.VMEM(((S - 1) * C,), jnp.float32),
        ],
    )
    return f(dens_t, z_t, rays_flat, u)


N_SC = 49152


@jax.jit
def _run(density2, z_vals, rays_d, u):
    n = density2.shape[0]
    n_sc = N_SC if n % 4096 == 0 and n > N_SC else 0
    n_tc = n - n_sc

    dens_t = density2.T
    z_t = z_vals.T

    out_tc_t = _run_tc(
        dens_t[:, :n_tc], z_t[:, :n_tc],
        jnp.concatenate(
            [rays_d[:n_tc], jnp.zeros((n_tc, 5), rays_d.dtype)], axis=1).T,
        u[:n_tc].T)
    parts = [out_tc_t.T]
    if n_sc:
        rays_flat = jnp.concatenate(
            [rays_d[n_tc:].reshape(-1), jnp.zeros((16,), rays_d.dtype)])
        parts.append(_run_sc(dens_t[:, n_tc:], z_t[:, n_tc:], rays_flat,
                             u[n_tc:]))
    return jnp.concatenate(parts, axis=0) if len(parts) > 1 else parts[0]


def kernel(density, z_vals, rays_d, u, N_importance):
    del N_importance
    return _run(density[..., 0], z_vals, rays_d, u)
